# baseline (speedup 1.0000x reference)
import functools

import jax
import jax.numpy as jnp
from jax import lax
from jax.experimental import pallas as pl
from jax.experimental.pallas import tpu as pltpu
from jax.experimental.pallas import tpu_sc as plsc

_NC = 2
_NS = 16
_NW = _NC * _NS
_L = 16
_NB = 4
_SUP = 6


def _build_table(x, W, b):
    N, D = x.shape
    R = W.shape[0]
    BN = 2000

    def body(x_ref, w_ref, b_ref, o_ref):
        z = jnp.dot(x_ref[...], w_ref[0], preferred_element_type=jnp.float32)
        z = z + b_ref[0]
        o_ref[0] = jnp.where(z > 0, z, jnp.exp(z) - 1.0)

    return pl.pallas_call(
        body,
        grid=(N // BN, R),
        in_specs=[
            pl.BlockSpec((BN, D), lambda i, r: (i, 0)),
            pl.BlockSpec((1, D, D), lambda i, r: (r, 0, 0)),
            pl.BlockSpec((1, 1, D), lambda i, r: (r, 0, 0)),
        ],
        out_specs=pl.BlockSpec((1, BN, D), lambda i, r: (r, i, 0)),
        out_shape=jax.ShapeDtypeStruct((R, N, D), jnp.float32),
    )(x, W, b.reshape(R, 1, D))


def _sc_aggregate(T, src, et, dst, zeros2, zeros1, N):
    RN, D = T.shape
    E = src.shape[0]
    EW = E // _NW
    K = 64
    FULL = EW // K
    TAIL = EW - FULL * K
    NP = zeros2.shape[0]
    ZR = NP // _NS
    NSUP = FULL // _SUP
    SE = _SUP * K
    assert FULL % _NB == 0 and FULL == NSUP * _SUP and NSUP % 2 == 0
    assert (2 * _SUP) % _NB == 0

    mesh = plsc.VectorSubcoreMesh(core_axis_name="c", subcore_axis_name="s")

    scratch = [
        [pltpu.VMEM((SE,), jnp.int32) for _ in range(2)],
        [pltpu.VMEM((SE,), jnp.int32) for _ in range(2)],
        [pltpu.VMEM((SE,), jnp.int32) for _ in range(2)],
        [pltpu.VMEM((K,), jnp.int32) for _ in range(_NB)],
        [pltpu.VMEM((K,), jnp.int32) for _ in range(_NB)],
        [pltpu.VMEM((K, D), jnp.float32) for _ in range(_NB)],
        pltpu.VMEM((K,), jnp.float32),
        pltpu.VMEM_SHARED((NP, D), jnp.float32),
        pltpu.VMEM_SHARED((NP,), jnp.float32),
        [pltpu.SemaphoreType.DMA for _ in range(2)],
        [pltpu.SemaphoreType.DMA for _ in range(_NB)],
        [pltpu.SemaphoreType.DMA for _ in range(_NB)],
        pltpu.SemaphoreType.DMA,
    ]
    if TAIL:
        scratch += [
            pltpu.VMEM((TAIL,), jnp.int32),
            pltpu.VMEM((TAIL,), jnp.int32),
            pltpu.VMEM((TAIL,), jnp.int32),
            pltpu.VMEM((TAIL,), jnp.int32),
            pltpu.VMEM((TAIL, D), jnp.float32),
            pltpu.VMEM((TAIL,), jnp.float32),
        ]

    @functools.partial(
        pl.kernel,
        out_type=(
            jax.ShapeDtypeStruct((_NC, NP, D), jnp.float32),
            jax.ShapeDtypeStruct((_NC, NP), jnp.float32),
        ),
        mesh=mesh,
        scratch_types=scratch,
    )
    def body(t_hbm, src_hbm, et_hbm, dst_hbm, zero2_hbm, zero1_hbm,
             out_hbm, outdeg_hbm,
             ssrc, setv, sdst, idxv, dstw, rows, onesv, agg_sh, deg_sh,
             stsem, gsem, ssem, msem, *tailbufs):
        c = lax.axis_index("c")
        s = lax.axis_index("s")
        wid = s * _NC + c
        base = wid * EW

        def start_staging(sg, g):
            off = base + g * SE
            pltpu.async_copy(src_hbm.at[pl.ds(off, SE)], ssrc[sg], stsem[sg])
            pltpu.async_copy(et_hbm.at[pl.ds(off, SE)], setv[sg], stsem[sg])
            pltpu.async_copy(dst_hbm.at[pl.ds(off, SE)], sdst[sg], stsem[sg])

        def wait_staging(sg, g):
            off = base + g * SE
            pltpu.make_async_copy(src_hbm.at[pl.ds(off, SE)], ssrc[sg],
                                  stsem[sg]).wait()
            pltpu.make_async_copy(et_hbm.at[pl.ds(off, SE)], setv[sg],
                                  stsem[sg]).wait()
            pltpu.make_async_copy(dst_hbm.at[pl.ds(off, SE)], sdst[sg],
                                  stsem[sg]).wait()

        def calc_idx(b, sg, j):
            for i in range(K // _L):
                sl = pl.ds(j * K + i * _L, _L)
                idxv[b][pl.ds(i * _L, _L)] = setv[sg][sl] * N + ssrc[sg][sl]

        def fill_dst(b, sg, j):
            for i in range(K // _L):
                sl = pl.ds(j * K + i * _L, _L)
                dstw[b][pl.ds(i * _L, _L)] = sdst[sg][sl]

        def start_gather(b):
            pltpu.async_copy(t_hbm.at[idxv[b]], rows[b], gsem[b])

        def wait_gather(b):
            pltpu.make_async_copy(t_hbm.at[idxv[b]], rows[b],
                                  gsem[b]).wait()

        def start_scatter(b):
            pltpu.async_copy(rows[b], agg_sh.at[dstw[b]], ssem[b], add=True)
            pltpu.async_copy(onesv, deg_sh.at[dstw[b]], ssem[b], add=True)

        def wait_scatter(b):
            pltpu.make_async_copy(rows[b], agg_sh.at[dstw[b]],
                                  ssem[b]).wait()
            pltpu.make_async_copy(onesv, deg_sh.at[dstw[b]],
                                  ssem[b]).wait()

        start_staging(0, 0)
        start_staging(1, 1)

        pltpu.sync_copy(zero2_hbm.at[pl.ds(s * ZR, ZR)],
                        agg_sh.at[pl.ds(s * ZR, ZR)])
        pltpu.sync_copy(zero1_hbm.at[pl.ds(s * ZR, ZR)],
                        deg_sh.at[pl.ds(s * ZR, ZR)])

        def onesfill(i, _):
            onesv[pl.ds(i * _L, _L)] = jnp.full((_L,), 1.0, jnp.float32)
            return 0
        lax.fori_loop(0, K // _L, onesfill, 0)

        wait_staging(0, 0)
        for b in (0, 1):
            calc_idx(b, 0, b)
            fill_dst(b, 0, b)
            start_gather(b)
        plsc.subcore_barrier()

        def pair(i, _):
            for su in range(2):
                g = 2 * i + su
                sg = su
                for j in range(_SUP):
                    cur = g * _SUP + j
                    q = (su * _SUP + j) % _NB
                    p = (q + 2) % _NB
                    j2 = j + 2
                    sg2 = sg if j2 < _SUP else 1 - sg
                    jw = j2 if j2 < _SUP else j2 - _SUP

                    wait_gather(q)
                    start_scatter(q)

                    @pl.when(cur + 2 < FULL)
                    def _():
                        if jw == 0:
                            wait_staging(sg2, g + 1)
                        calc_idx(p, sg2, jw)

                    @pl.when(cur >= 2)
                    def _():
                        wait_scatter(p)

                    @pl.when(cur + 2 < FULL)
                    def _():
                        fill_dst(p, sg2, jw)
                        start_gather(p)

                @pl.when(g + 2 < NSUP)
                def _():
                    start_staging(sg, g + 2)
            return 0
        lax.fori_loop(0, NSUP // 2, pair, 0)
        wait_scatter((FULL - 2) % _NB)
        wait_scatter((FULL - 1) % _NB)

        if TAIL:
            src_t, et_t, idx_t, dst_t, rows_t, ones_t = tailbufs
            tb = base + FULL * K
            pltpu.sync_copy(src_hbm.at[pl.ds(tb, TAIL)], src_t)
            pltpu.sync_copy(et_hbm.at[pl.ds(tb, TAIL)], et_t)
            pltpu.sync_copy(dst_hbm.at[pl.ds(tb, TAIL)], dst_t)

            def onet(i, _):
                sl = pl.ds(i * _L, _L)
                idx_t[sl] = et_t[sl] * N + src_t[sl]
                ones_t[sl] = jnp.full((_L,), 1.0, jnp.float32)
                return 0
            lax.fori_loop(0, TAIL // _L, onet, 0)
            pltpu.async_copy(t_hbm.at[idx_t], rows_t, msem)
            pltpu.make_async_copy(t_hbm.at[idx_t], rows_t, msem).wait()
            pltpu.sync_copy(rows_t, agg_sh.at[dst_t], add=True)
            pltpu.sync_copy(ones_t, deg_sh.at[dst_t], add=True)

        plsc.subcore_barrier()
        pltpu.sync_copy(agg_sh.at[pl.ds(s * ZR, ZR)],
                        out_hbm.at[c].at[pl.ds(s * ZR, ZR)])
        pltpu.sync_copy(deg_sh.at[pl.ds(s * ZR, ZR)],
                        outdeg_hbm.at[c].at[pl.ds(s * ZR, ZR)])

    return body(T, src, et, dst, zeros2, zeros1)


def _finalize(parts, pdeg3, N, D):
    NC, NP, _ = parts.shape
    BN = 2000

    def body(p_ref, d_ref, o_ref):
        ssum = p_ref[0] + p_ref[1]
        deg = jnp.maximum(d_ref[0] + d_ref[1], 1.0)
        w = ssum / deg
        o_ref[...] = jnp.where(w > 0, w, jnp.exp(w) - 1.0)

    return pl.pallas_call(
        body,
        grid=(N // BN,),
        in_specs=[
            pl.BlockSpec((NC, BN, D), lambda i: (0, i, 0)),
            pl.BlockSpec((NC, BN, 1), lambda i: (0, i, 0)),
        ],
        out_specs=pl.BlockSpec((BN, D), lambda i: (i, 0)),
        out_shape=jax.ShapeDtypeStruct((N, D), jnp.float32),
    )(parts, pdeg3)


def kernel(x, edge_index, edge_type, W, b):
    N, D = x.shape
    R = W.shape[0]
    T = _build_table(x, W, b).reshape(R * N, D)
    src = edge_index[0]
    dst = edge_index[1]
    NP = ((N + 128 * _NS - 1) // (128 * _NS)) * (128 * _NS)
    zeros2 = jnp.zeros((NP, D), jnp.float32)
    zeros1 = jnp.zeros((NP,), jnp.float32)
    parts, pdeg = _sc_aggregate(T, src, edge_type, dst, zeros2, zeros1, N)
    return _finalize(parts, pdeg[:, :, None], N, D)


# speedup vs baseline: 1.0008x; 1.0008x over previous
import functools

import jax
import jax.numpy as jnp
from jax import lax
from jax.experimental import pallas as pl
from jax.experimental.pallas import tpu as pltpu
from jax.experimental.pallas import tpu_sc as plsc

_NC = 2
_NS = 16
_NW = _NC * _NS
_L = 16
_NB = 4
_SUP = 6


def _build_table(x, W, b):
    N, D = x.shape
    R = W.shape[0]
    BN = 2000

    def body(x_ref, w_ref, b_ref, o_ref):
        z = jnp.dot(x_ref[...], w_ref[0], preferred_element_type=jnp.float32)
        z = z + b_ref[0]
        o_ref[0] = jnp.where(z > 0, z, jnp.exp(z) - 1.0)

    return pl.pallas_call(
        body,
        grid=(N // BN, R),
        in_specs=[
            pl.BlockSpec((BN, D), lambda i, r: (i, 0)),
            pl.BlockSpec((1, D, D), lambda i, r: (r, 0, 0)),
            pl.BlockSpec((1, 1, D), lambda i, r: (r, 0, 0)),
        ],
        out_specs=pl.BlockSpec((1, BN, D), lambda i, r: (r, i, 0)),
        out_shape=jax.ShapeDtypeStruct((R, N, D), jnp.float32),
    )(x, W, b.reshape(R, 1, D))


def _sc_aggregate(T, src, et, dst, zeros2, zeros1, N):
    RN, D = T.shape
    E = src.shape[0]
    EW = E // _NW
    K = 64
    FULL = EW // K
    TAIL = EW - FULL * K
    NP = zeros2.shape[0]
    ZR = NP // _NS
    NSUP = FULL // _SUP
    SE = _SUP * K
    assert FULL % _NB == 0 and FULL == NSUP * _SUP and NSUP % 2 == 0
    assert (2 * _SUP) % _NB == 0

    mesh = plsc.VectorSubcoreMesh(core_axis_name="c", subcore_axis_name="s")

    scratch = [
        [pltpu.VMEM((SE,), jnp.int32) for _ in range(2)],
        [pltpu.VMEM((SE,), jnp.int32) for _ in range(2)],
        [pltpu.VMEM((SE,), jnp.int32) for _ in range(2)],
        [pltpu.VMEM((K,), jnp.int32) for _ in range(_NB)],
        [pltpu.VMEM((K,), jnp.int32) for _ in range(_NB)],
        [pltpu.VMEM((K, D), jnp.float32) for _ in range(_NB)],
        pltpu.VMEM((K,), jnp.float32),
        pltpu.VMEM_SHARED((NP, D), jnp.float32),
        pltpu.VMEM_SHARED((NP,), jnp.float32),
        [pltpu.SemaphoreType.DMA for _ in range(2)],
        [pltpu.SemaphoreType.DMA for _ in range(_NB)],
        [pltpu.SemaphoreType.DMA for _ in range(_NB)],
        pltpu.SemaphoreType.DMA,
    ]
    if TAIL:
        scratch += [
            pltpu.VMEM((TAIL,), jnp.int32),
            pltpu.VMEM((TAIL,), jnp.int32),
            pltpu.VMEM((TAIL,), jnp.int32),
            pltpu.VMEM((TAIL,), jnp.int32),
            pltpu.VMEM((TAIL, D), jnp.float32),
            pltpu.VMEM((TAIL,), jnp.float32),
        ]

    @functools.partial(
        pl.kernel,
        out_type=(
            jax.ShapeDtypeStruct((_NC, NP, D), jnp.float32),
            jax.ShapeDtypeStruct((_NC, NP), jnp.float32),
        ),
        mesh=mesh,
        scratch_types=scratch,
    )
    def body(t_hbm, src_hbm, et_hbm, dst_hbm, zero2_hbm, zero1_hbm,
             out_hbm, outdeg_hbm,
             ssrc, setv, sdst, idxv, dstw, rows, onesv, agg_sh, deg_sh,
             stsem, gsem, ssem, msem, *tailbufs):
        c = lax.axis_index("c")
        s = lax.axis_index("s")
        wid = s * _NC + c
        base = wid * EW

        def start_staging(sg, g):
            off = base + g * SE
            pltpu.async_copy(src_hbm.at[pl.ds(off, SE)], ssrc[sg], stsem[sg])
            pltpu.async_copy(et_hbm.at[pl.ds(off, SE)], setv[sg], stsem[sg])
            pltpu.async_copy(dst_hbm.at[pl.ds(off, SE)], sdst[sg], stsem[sg])

        def wait_staging(sg, g):
            off = base + g * SE
            pltpu.make_async_copy(src_hbm.at[pl.ds(off, SE)], ssrc[sg],
                                  stsem[sg]).wait()
            pltpu.make_async_copy(et_hbm.at[pl.ds(off, SE)], setv[sg],
                                  stsem[sg]).wait()
            pltpu.make_async_copy(dst_hbm.at[pl.ds(off, SE)], sdst[sg],
                                  stsem[sg]).wait()

        def calc_idx(b, sg, j):
            for i in range(K // _L):
                sl = pl.ds(j * K + i * _L, _L)
                idxv[b][pl.ds(i * _L, _L)] = setv[sg][sl] * N + ssrc[sg][sl]

        def fill_dst(b, sg, j):
            for i in range(K // _L):
                sl = pl.ds(j * K + i * _L, _L)
                dstw[b][pl.ds(i * _L, _L)] = sdst[sg][sl]

        def start_gather(b):
            pltpu.async_copy(t_hbm.at[idxv[b]], rows[b], gsem[b])

        def wait_gather(b):
            pltpu.make_async_copy(t_hbm.at[idxv[b]], rows[b],
                                  gsem[b]).wait()

        def start_scatter(b):
            pltpu.async_copy(rows[b], agg_sh.at[dstw[b]], ssem[b], add=True)
            pltpu.async_copy(onesv, deg_sh.at[dstw[b]], ssem[b], add=True)

        def wait_scatter(b):
            pltpu.make_async_copy(rows[b], agg_sh.at[dstw[b]],
                                  ssem[b]).wait()
            pltpu.make_async_copy(onesv, deg_sh.at[dstw[b]],
                                  ssem[b]).wait()

        start_staging(0, 0)
        start_staging(1, 1)

        pltpu.sync_copy(zero2_hbm.at[pl.ds(s * ZR, ZR)],
                        agg_sh.at[pl.ds(s * ZR, ZR)])
        pltpu.sync_copy(zero1_hbm.at[pl.ds(s * ZR, ZR)],
                        deg_sh.at[pl.ds(s * ZR, ZR)])

        def onesfill(i, _):
            onesv[pl.ds(i * _L, _L)] = jnp.full((_L,), 1.0, jnp.float32)
            return 0
        lax.fori_loop(0, K // _L, onesfill, 0)

        wait_staging(0, 0)
        for b in (0, 1):
            calc_idx(b, 0, b)
            fill_dst(b, 0, b)
            start_gather(b)
        plsc.subcore_barrier()

        def pair(i, _):
            for su in range(2):
                g = 2 * i + su
                sg = su
                for j in range(_SUP):
                    cur = g * _SUP + j
                    q = (su * _SUP + j) % _NB
                    p = (q + 2) % _NB
                    j2 = j + 2
                    sg2 = sg if j2 < _SUP else 1 - sg
                    jw = j2 if j2 < _SUP else j2 - _SUP

                    wait_gather(q)
                    start_scatter(q)

                    @pl.when(cur >= 2)
                    def _():
                        wait_scatter(p)

                    @pl.when(cur + 2 < FULL)
                    def _():
                        if jw == 0:
                            wait_staging(sg2, g + 1)
                        calc_idx(p, sg2, jw)
                        fill_dst(p, sg2, jw)
                        start_gather(p)

                @pl.when(g + 2 < NSUP)
                def _():
                    start_staging(sg, g + 2)
            return 0
        lax.fori_loop(0, NSUP // 2, pair, 0)
        wait_scatter((FULL - 2) % _NB)
        wait_scatter((FULL - 1) % _NB)

        if TAIL:
            src_t, et_t, idx_t, dst_t, rows_t, ones_t = tailbufs
            tb = base + FULL * K
            pltpu.sync_copy(src_hbm.at[pl.ds(tb, TAIL)], src_t)
            pltpu.sync_copy(et_hbm.at[pl.ds(tb, TAIL)], et_t)
            pltpu.sync_copy(dst_hbm.at[pl.ds(tb, TAIL)], dst_t)

            def onet(i, _):
                sl = pl.ds(i * _L, _L)
                idx_t[sl] = et_t[sl] * N + src_t[sl]
                ones_t[sl] = jnp.full((_L,), 1.0, jnp.float32)
                return 0
            lax.fori_loop(0, TAIL // _L, onet, 0)
            pltpu.async_copy(t_hbm.at[idx_t], rows_t, msem)
            pltpu.make_async_copy(t_hbm.at[idx_t], rows_t, msem).wait()
            pltpu.sync_copy(rows_t, agg_sh.at[dst_t], add=True)
            pltpu.sync_copy(ones_t, deg_sh.at[dst_t], add=True)

        plsc.subcore_barrier()
        pltpu.sync_copy(agg_sh.at[pl.ds(s * ZR, ZR)],
                        out_hbm.at[c].at[pl.ds(s * ZR, ZR)])
        pltpu.sync_copy(deg_sh.at[pl.ds(s * ZR, ZR)],
                        outdeg_hbm.at[c].at[pl.ds(s * ZR, ZR)])

    return body(T, src, et, dst, zeros2, zeros1)


def _finalize(parts, pdeg3, N, D):
    NC, NP, _ = parts.shape
    BN = 2000

    def body(p_ref, d_ref, o_ref):
        ssum = p_ref[0] + p_ref[1]
        deg = jnp.maximum(d_ref[0] + d_ref[1], 1.0)
        w = ssum / deg
        o_ref[...] = jnp.where(w > 0, w, jnp.exp(w) - 1.0)

    return pl.pallas_call(
        body,
        grid=(N // BN,),
        in_specs=[
            pl.BlockSpec((NC, BN, D), lambda i: (0, i, 0)),
            pl.BlockSpec((NC, BN, 1), lambda i: (0, i, 0)),
        ],
        out_specs=pl.BlockSpec((BN, D), lambda i: (i, 0)),
        out_shape=jax.ShapeDtypeStruct((N, D), jnp.float32),
    )(parts, pdeg3)


def kernel(x, edge_index, edge_type, W, b):
    N, D = x.shape
    R = W.shape[0]
    T = _build_table(x, W, b).reshape(R * N, D)
    src = edge_index[0]
    dst = edge_index[1]
    NP = ((N + 128 * _NS - 1) // (128 * _NS)) * (128 * _NS)
    zeros2 = jnp.zeros((NP, D), jnp.float32)
    zeros1 = jnp.zeros((NP,), jnp.float32)
    parts, pdeg = _sc_aggregate(T, src, edge_type, dst, zeros2, zeros1, N)
    return _finalize(parts, pdeg[:, :, None], N, D)
